# Initial kernel scaffold; baseline (speedup 1.0000x reference)
import jax
import jax.numpy as jnp
from jax.experimental import pallas as pl


def kernel(str_id, num_modifications, random_replacements, W, num_samples):
    raise NotImplementedError("write your pallas kernel here")


# speedup vs baseline: 136.6076x; 136.6076x over previous
import functools

import jax
import jax.numpy as jnp
from jax import lax
from jax.experimental import pallas as pl
from jax.experimental.pallas import tpu as pltpu
from jax.experimental.pallas import tpu_sc as plsc

MAXD = 100000
L = 20
BATCH = 5 * MAXD
NW = 32
K = 32
C = 512
NCH = 30
R_MAIN = 15632
R_LAST = BATCH - 31 * R_MAIN
TAIL = R_MAIN - NCH * C

_mesh = plsc.VectorSubcoreMesh(core_axis_name="c", subcore_axis_name="s")


def _iota16():
    return lax.iota(jnp.int32, 16)


def _merge16(tv, ti, th, vals, idx):
    sp, si = plsc.sort_key_val(vals, idx, descending=True)
    v0 = tv[pl.ds(0, 16)]
    v1 = tv[pl.ds(16, 16)]
    i0 = ti[pl.ds(0, 16)]
    i1 = ti[pl.ds(16, 16)]
    rc = lax.rev(sp, (0,))
    rci = lax.rev(si, (0,))
    take = (v1 > rc) | ((v1 == rc) & (i1 < rci))
    a = jnp.where(take, v1, rc)
    ai = jnp.where(take, i1, rci)
    a, ai = plsc.sort_key_val(a, ai, descending=True)
    ra = lax.rev(a, (0,))
    rai = lax.rev(ai, (0,))
    take2 = (v0 > ra) | ((v0 == ra) & (i0 < rai))
    p = jnp.where(take2, v0, ra)
    pi = jnp.where(take2, i0, rai)
    q = jnp.where(take2, ra, v0)
    qi = jnp.where(take2, rai, i0)
    nv0, ni0 = plsc.sort_key_val(p, pi, descending=True)
    nv1, ni1 = plsc.sort_key_val(q, qi, descending=True)
    tv[pl.ds(0, 16)] = nv0
    tv[pl.ds(16, 16)] = nv1
    ti[pl.ds(0, 16)] = ni0
    ti[pl.ds(16, 16)] = ni1
    th[...] = jnp.broadcast_to(jnp.min(nv1), (16,))


def _p1_body(rr_hbm, nm_hbm, sid_hbm, w_hbm, ov_hbm, oi_hbm,
             w_vmem, chunk0, chunk1, nm0, nm1, tchunk, tnm,
             sidb, sfx, tv, ti, th, s0, s1):
    nc = 2
    wid = lax.axis_index("s") * nc + lax.axis_index("c")
    base = wid * R_MAIN
    iota = _iota16()
    iota20 = iota * 20

    pltpu.sync_copy(w_hbm, w_vmem)
    pltpu.sync_copy(sid_hbm, sidb)

    b_lo = plsc.load_gather(w_vmem, [sidb[pl.ds(0, 16)]])
    b_hi = plsc.load_gather(w_vmem, [sidb[pl.ds(16, 16)]])
    b_hi = jnp.where(iota < 4, b_hi, jnp.float32(0.0))
    hi_sum = jnp.sum(b_hi)
    sfx_hi = lax.rev(plsc.cumsum(lax.rev(b_hi, (0,))), (0,))
    sfx_lo = lax.rev(plsc.cumsum(lax.rev(b_lo, (0,))), (0,)) + hi_sum
    sfx[pl.ds(0, 16)] = sfx_lo
    sfx[pl.ds(16, 16)] = sfx_hi

    ninf = jnp.full((16,), -jnp.inf, jnp.float32)
    zero16 = jnp.zeros((16,), jnp.int32)
    tv[pl.ds(0, 16)] = ninf
    tv[pl.ds(16, 16)] = ninf
    ti[pl.ds(0, 16)] = zero16
    ti[pl.ds(16, 16)] = zero16
    th[...] = ninf

    sems = (s0, s1)
    chunks = (chunk0, chunk1)
    nms = (nm0, nm1)

    def fetch(c, b):
        r0 = base + c * C
        pltpu.async_copy(rr_hbm.at[pl.ds(r0 * L, C * L)], chunks[b], sems[b])
        pltpu.async_copy(nm_hbm.at[pl.ds(r0, C)], nms[b], sems[b])

    def wait_buf(b):
        pltpu.make_async_copy(rr_hbm.at[pl.ds(0, C * L)], chunks[b], sems[b]).wait()
        pltpu.make_async_copy(nm_hbm.at[pl.ds(0, C)], nms[b], sems[b]).wait()

    def make_grp(ch, nm_ref, row0, min_row=None):
        def grp(g):
            m = nm_ref[pl.ds(g * 16, 16)]
            fi = g * 320 + iota20
            acc = jnp.zeros((16,), jnp.float32)
            for l in range(L):
                tok = plsc.load_gather(ch, [fi + l])
                wv = plsc.load_gather(w_vmem, [tok])
                acc = acc + jnp.where(l < m, wv, jnp.float32(0.0))
            acc = acc + plsc.load_gather(sfx, [m])
            gidx = row0 + g * 16 + iota
            if min_row is not None:
                acc = jnp.where(gidx >= min_row, acc, -jnp.inf)
            hit = jnp.any(acc > th[...])

            @pl.when(hit)
            def _():
                _merge16(tv, ti, th, acc, gidx)

        return grp

    fetch(0, 0)
    fetch(1, 1)

    def outer(cc):
        for b in range(2):
            c = cc + b
            wait_buf(b)
            pl.loop(0, C // 16)(make_grp(chunks[b], nms[b], base + c * C))
            nxt = c + 2

            @pl.when(nxt < NCH)
            def _():
                fetch(nxt, b)

    pl.loop(0, NCH, step=2)(outer)

    tail0 = base + NCH * C
    tfetch = jnp.minimum(tail0, BATCH - TAIL)
    pltpu.sync_copy(rr_hbm.at[pl.ds(tfetch * L, TAIL * L)], tchunk)
    pltpu.sync_copy(nm_hbm.at[pl.ds(tfetch, TAIL)], tnm)
    pl.loop(0, TAIL // 16)(make_grp(tchunk, tnm, tfetch, min_row=tail0))

    pltpu.sync_copy(tv, ov_hbm.at[wid])
    pltpu.sync_copy(ti, oi_hbm.at[wid])


@jax.jit
def _phase1(rr_flat, nm, sid_pad, w):
    return pl.kernel(
        _p1_body,
        out_type=(
            jax.ShapeDtypeStruct((NW, K), jnp.float32),
            jax.ShapeDtypeStruct((NW, K), jnp.int32),
        ),
        mesh=_mesh,
        compiler_params=pltpu.CompilerParams(needs_layout_passes=False),
        scratch_types=[
            pltpu.VMEM((MAXD,), jnp.float32),
            pltpu.VMEM((C * L,), jnp.int32),
            pltpu.VMEM((C * L,), jnp.int32),
            pltpu.VMEM((C,), jnp.int32),
            pltpu.VMEM((C,), jnp.int32),
            pltpu.VMEM((TAIL * L,), jnp.int32),
            pltpu.VMEM((TAIL,), jnp.int32),
            pltpu.VMEM((32,), jnp.int32),
            pltpu.VMEM((32,), jnp.float32),
            pltpu.VMEM((32,), jnp.float32),
            pltpu.VMEM((32,), jnp.int32),
            pltpu.VMEM((16,), jnp.float32),
            pltpu.SemaphoreType.DMA,
            pltpu.SemaphoreType.DMA,
        ],
    )(rr_flat, nm, sid_pad, w)


def _p2_body(cv_hbm, ci_hbm, rr_hbm, nm_hbm, sid_hbm, oi_hbm, ov_hbm,
             cvb, cib, idxb, rowbuf, nmbuf, sidb, oib, ovb, sem):
    nc = 2
    wid = lax.axis_index("s") * nc + lax.axis_index("c")

    @pl.when(wid == 0)
    def _():
        iota = _iota16()
        pltpu.sync_copy(cv_hbm, cvb)
        pltpu.sync_copy(ci_hbm, cib)
        pltpu.sync_copy(sid_hbm, sidb)
        v0 = cvb[pl.ds(0, 16)]
        v1 = cvb[pl.ds(16, 16)]
        i0 = cib[pl.ds(0, 16)]
        i1 = cib[pl.ds(16, 16)]
        for w in range(1, NW):
            c0 = cvb[pl.ds(32 * w, 16)]
            c1 = cvb[pl.ds(32 * w + 16, 16)]
            d0 = cib[pl.ds(32 * w, 16)]
            d1 = cib[pl.ds(32 * w + 16, 16)]
            rc1 = lax.rev(c1, (0,))
            rd1 = lax.rev(d1, (0,))
            rc0 = lax.rev(c0, (0,))
            rd0 = lax.rev(d0, (0,))
            t1 = (v0 > rc1) | ((v0 == rc1) & (i0 < rd1))
            p1 = jnp.where(t1, v0, rc1)
            p1i = jnp.where(t1, i0, rd1)
            t2 = (v1 > rc0) | ((v1 == rc0) & (i1 < rd0))
            p2 = jnp.where(t2, v1, rc0)
            p2i = jnp.where(t2, i1, rd0)
            t3 = (p1 > p2) | ((p1 == p2) & (p1i < p2i))
            hi = jnp.where(t3, p1, p2)
            hii = jnp.where(t3, p1i, p2i)
            lo = jnp.where(t3, p2, p1)
            loi = jnp.where(t3, p2i, p1i)
            v0, i0 = plsc.sort_key_val(hi, hii, descending=True)
            v1, i1 = plsc.sort_key_val(lo, loi, descending=True)
        ovb[pl.ds(0, 16)] = v0
        ovb[pl.ds(16, 16)] = v1
        idxb[pl.ds(0, 16)] = i0
        idxb[pl.ds(16, 16)] = i1
        pltpu.sync_copy(ovb, ov_hbm)
        cps = []
        ds = []
        dns = []
        for i in range(K):
            idx = i0[i] if i < 16 else i1[i - 16]
            off = idx * L
            a = pl.multiple_of(jnp.minimum(off - (off & 7), BATCH * L - 128), 8)
            an = pl.multiple_of(jnp.minimum(idx - (idx & 7), BATCH - 128), 8)
            ds.append(off - a)
            dns.append(idx - an)
            cps.append(pltpu.async_copy(
                rr_hbm.at[pl.ds(a, 128)],
                rowbuf.at[pl.ds(128 * i, 128)], sem))
            cps.append(pltpu.async_copy(
                nm_hbm.at[pl.ds(an, 128)],
                nmbuf.at[pl.ds(128 * i, 128)], sem))
        for cp in cps:
            cp.wait()
        sid_lo = sidb[pl.ds(0, 16)]
        sid_hi = sidb[pl.ds(16, 16)]
        for i in range(K):
            mi = plsc.load_gather(
                nmbuf, [jnp.broadcast_to(128 * i + dns[i], (16,))])
            rlo = plsc.load_gather(rowbuf, [128 * i + ds[i] + iota])
            out_lo = jnp.where(iota < mi, rlo, sid_lo)
            plsc.store_scatter(oib, [L * i + iota], out_lo)
            chi = 16 + iota
            rhi = plsc.load_gather(
                rowbuf, [jnp.minimum(128 * i + ds[i] + chi, 128 * i + 127)])
            out_hi = jnp.where(chi < mi, rhi, sid_hi)
            plsc.store_scatter(oib, [jnp.minimum(L * i + chi, K * L - 1)],
                               out_hi, mask=iota < 4)
        pltpu.sync_copy(oib, oi_hbm)


@jax.jit
def _phase2(cand_v, cand_i, rr_flat, nm, sid_pad):
    return pl.kernel(
        _p2_body,
        out_type=(
            jax.ShapeDtypeStruct((K * L,), jnp.int32),
            jax.ShapeDtypeStruct((K,), jnp.float32),
        ),
        mesh=_mesh,
        compiler_params=pltpu.CompilerParams(needs_layout_passes=False),
        scratch_types=[
            pltpu.VMEM((NW * K,), jnp.float32),
            pltpu.VMEM((NW * K,), jnp.int32),
            pltpu.VMEM((K,), jnp.int32),
            pltpu.VMEM((K * 128,), jnp.int32),
            pltpu.VMEM((K * 128,), jnp.int32),
            pltpu.VMEM((32,), jnp.int32),
            pltpu.VMEM((K * L,), jnp.int32),
            pltpu.VMEM((K,), jnp.float32),
            pltpu.SemaphoreType.DMA,
        ],
    )(cand_v.reshape(-1), cand_i.reshape(-1), rr_flat, nm, sid_pad)


def kernel(str_id, num_modifications, random_replacements, W, num_samples):
    rr_flat = random_replacements.reshape(-1)
    sid_pad = jnp.zeros((32,), jnp.int32).at[:L].set(str_id[0])
    cand_v, cand_i = _phase1(rr_flat, num_modifications, sid_pad, W)
    top_inputs, top_vals = _phase2(cand_v, cand_i, rr_flat,
                                   num_modifications, sid_pad)
    return top_inputs.reshape(K, L), top_vals
